# baseline (speedup 1.0000x reference)
import functools

import jax
import jax.numpy as jnp
from jax import lax
from jax.experimental import pallas as pl
from jax.experimental.pallas import tpu as pltpu
from jax.experimental.pallas import tpu_sc as plsc

N = 10000
E = 160000
DIN = 256
DH = 512
DOUT = 128

DC = 128
NC = 2
NS = 16
EB = 128
SB = 40
NB = 80
NSB = NB // SB
EPT = NB * EB
EPAD = NS * EPT
RPT = 640
NPAD = RPT * NS
TAIL = N - RPT * (NS - 1)
TRASH = N

BN = 1000
GN = N // BN


def _make_seg_sum(C):
    cpc = C // NC
    mesh = plsc.VectorSubcoreMesh(core_axis_name="c", subcore_axis_name="s",
                                  num_cores=NC, num_subcores=NS)

    @functools.partial(
        pl.kernel,
        out_type=jax.ShapeDtypeStruct((C, N, DC), jnp.float32),
        mesh=mesh,
        scratch_types=[
            pltpu.VMEM((SB, EB), jnp.int32),
            pltpu.VMEM((SB, EB), jnp.int32),
            pltpu.VMEM((EB,), jnp.int32),
            pltpu.VMEM((EB,), jnp.int32),
            pltpu.VMEM((EB,), jnp.int32),
            pltpu.VMEM((EB,), jnp.int32),
            pltpu.VMEM((EB, DC), jnp.float32),
            pltpu.VMEM((EB, DC), jnp.float32),
            pltpu.VMEM_SHARED((NPAD, DC), jnp.float32),
            pltpu.SemaphoreType.DMA,
            pltpu.SemaphoreType.DMA,
        ],
    )
    def seg(x_hbm, src_hbm, dst_hbm, zeros_hbm, out_hbm,
            src_v, dst_v, sidx_a, sidx_b, didx_a, didx_b,
            rows_a, rows_b, agg_sh, gsem_a, gsem_b):
        cid = lax.axis_index("c")
        sid = lax.axis_index("s")
        full = pl.ds(sid * RPT, RPT)
        tail = pl.ds((NS - 1) * RPT, TAIL)
        sidx = (sidx_a, sidx_b)
        didx = (didx_a, didx_b)
        rows = (rows_a, rows_b)
        gsem = (gsem_a, gsem_b)

        def prep_gather(k, p, c):
            for q in range(EB // 16):
                sl = pl.ds(q * 16, 16)
                sidx[p][sl] = src_v[k, sl] * C + c
            pltpu.async_copy(x_hbm.at[sidx[p]], rows[p], gsem[p])

        def wait_gather(p):
            pltpu.make_async_copy(x_hbm.at[sidx[p]], rows[p], gsem[p]).wait()

        def scatter(k, p):
            for q in range(EB // 16):
                sl = pl.ds(q * 16, 16)
                didx[p][sl] = dst_v[k, sl]
            pltpu.sync_copy(rows[p], agg_sh.at[didx[p]], add=True)

        for cc in range(cpc):
            c = cid * cpc + cc

            @pl.when(sid < NS - 1)
            def _():
                pltpu.sync_copy(zeros_hbm, agg_sh.at[full])

            @pl.when(sid == NS - 1)
            def _():
                pltpu.sync_copy(zeros_hbm.at[pl.ds(0, TAIL)], agg_sh.at[tail])

            plsc.subcore_barrier()

            for g in range(NSB):
                pltpu.sync_copy(src_hbm.at[sid, g], src_v)
                pltpu.sync_copy(dst_hbm.at[sid, g], dst_v)
                prep_gather(0, 0, c)
                prep_gather(1, 1, c)

                def pair(jj, carry):
                    j0 = jj * 2
                    wait_gather(0)
                    scatter(j0, 0)
                    prep_gather(j0 + 2, 0, c)
                    wait_gather(1)
                    scatter(j0 + 1, 1)
                    prep_gather(j0 + 3, 1, c)
                    return carry

                lax.fori_loop(0, SB // 2 - 1, pair, 0)
                wait_gather(0)
                scatter(SB - 2, 0)
                wait_gather(1)
                scatter(SB - 1, 1)

            plsc.subcore_barrier()

            @pl.when(sid < NS - 1)
            def _():
                pltpu.sync_copy(agg_sh.at[full], out_hbm.at[c].at[full])

            @pl.when(sid == NS - 1)
            def _():
                pltpu.sync_copy(agg_sh.at[tail], out_hbm.at[c].at[tail])

            plsc.subcore_barrier()

    return seg


_seg2 = _make_seg_sum(2)
_seg4 = _make_seg_sum(4)


def _make_mlp_a(C, din):
    def body(x_ref, agg_ref, w_ref, b_ref, h_ref, s1_ref, s2_ref):
        i = pl.program_id(0)
        agg = jnp.concatenate([agg_ref[c] for c in range(C)], axis=-1)
        xa = x_ref[...] + agg
        h = jnp.dot(xa, w_ref[...], preferred_element_type=jnp.float32)
        h = h + b_ref[...]
        h_ref[...] = h

        @pl.when(i == 0)
        def _():
            s1_ref[...] = jnp.zeros_like(s1_ref)
            s2_ref[...] = jnp.zeros_like(s2_ref)

        s1_ref[...] += jnp.sum(h, axis=0, keepdims=True)
        s2_ref[...] += jnp.sum(h * h, axis=0, keepdims=True)

    return pl.pallas_call(
        body,
        grid=(GN,),
        in_specs=[
            pl.BlockSpec((BN, din), lambda i: (i, 0)),
            pl.BlockSpec((C, BN, DC), lambda i: (0, i, 0)),
            pl.BlockSpec((din, DH), lambda i: (0, 0)),
            pl.BlockSpec((1, DH), lambda i: (0, 0)),
        ],
        out_specs=[
            pl.BlockSpec((BN, DH), lambda i: (i, 0)),
            pl.BlockSpec((1, DH), lambda i: (0, 0)),
            pl.BlockSpec((1, DH), lambda i: (0, 0)),
        ],
        out_shape=[
            jax.ShapeDtypeStruct((N, DH), jnp.float32),
            jax.ShapeDtypeStruct((1, DH), jnp.float32),
            jax.ShapeDtypeStruct((1, DH), jnp.float32),
        ],
    )


def _mlp_b_body(h_ref, s1_ref, s2_ref, g_ref, be_ref, w_ref, b_ref, o_ref):
    mu = s1_ref[...] / N
    var = s2_ref[...] / N - mu * mu
    hn = (h_ref[...] - mu) * lax.rsqrt(var + 1e-5) * g_ref[...] + be_ref[...]
    hn = jnp.maximum(hn, 0.0)
    o = jnp.dot(hn, w_ref[...], preferred_element_type=jnp.float32)
    o_ref[...] = jnp.maximum(o + b_ref[...], 0.0)


_mlp_b = pl.pallas_call(
    _mlp_b_body,
    grid=(GN,),
    in_specs=[
        pl.BlockSpec((BN, DH), lambda i: (i, 0)),
        pl.BlockSpec((1, DH), lambda i: (0, 0)),
        pl.BlockSpec((1, DH), lambda i: (0, 0)),
        pl.BlockSpec((1, DH), lambda i: (0, 0)),
        pl.BlockSpec((1, DH), lambda i: (0, 0)),
        pl.BlockSpec((DH, DH), lambda i: (0, 0)),
        pl.BlockSpec((1, DH), lambda i: (0, 0)),
    ],
    out_specs=pl.BlockSpec((BN, DH), lambda i: (i, 0)),
    out_shape=jax.ShapeDtypeStruct((N, DH), jnp.float32),
)


def _final_body(h1_ref, h2_ref, h3_ref, w_ref, b_ref, o_ref):
    hcat = jnp.concatenate([h1_ref[...], h2_ref[...], h3_ref[...]], axis=-1)
    acc = jnp.dot(hcat, w_ref[...], preferred_element_type=jnp.float32)
    acc = acc + b_ref[...]
    m = jnp.max(acc, axis=1, keepdims=True)
    s = jnp.sum(jnp.exp(acc - m), axis=1, keepdims=True)
    o_ref[...] = acc - m - jnp.log(s)


_final = pl.pallas_call(
    _final_body,
    grid=(GN,),
    in_specs=[
        pl.BlockSpec((BN, DH), lambda i: (i, 0)),
        pl.BlockSpec((BN, DH), lambda i: (i, 0)),
        pl.BlockSpec((BN, DH), lambda i: (i, 0)),
        pl.BlockSpec((3 * DH, DOUT), lambda i: (0, 0)),
        pl.BlockSpec((1, DOUT), lambda i: (0, 0)),
    ],
    out_specs=pl.BlockSpec((BN, DOUT), lambda i: (i, 0)),
    out_shape=jax.ShapeDtypeStruct((N, DOUT), jnp.float32),
)

_mlp_a2 = _make_mlp_a(2, DIN)
_mlp_a4 = _make_mlp_a(4, DH)


def kernel(x, edge_index, c1_W1, c1_b1, c1_g, c1_be, c1_W2, c1_b2,
           c2_W1, c2_b1, c2_g, c2_be, c2_W2, c2_b2,
           c3_W1, c3_b1, c3_g, c3_be, c3_W2, c3_b2, lin_W, lin_b):
    pad = EPAD - E
    src = jnp.concatenate(
        [edge_index[0], jnp.zeros((pad,), jnp.int32)]
    ).reshape(NS, NSB, SB, EB)
    dst = jnp.concatenate(
        [edge_index[1], jnp.full((pad,), TRASH, jnp.int32)]
    ).reshape(NS, NSB, SB, EB)
    zeros = jnp.zeros((RPT, DC), jnp.float32)
    r = lambda v: v.reshape(1, -1)

    agg1 = _seg2(x.reshape(N * 2, DC), src, dst, zeros)
    h0, s1, s2 = _mlp_a2(x, agg1, c1_W1, r(c1_b1))
    h1 = _mlp_b(h0, s1, s2, r(c1_g), r(c1_be), c1_W2, r(c1_b2))

    agg2 = _seg4(h1.reshape(N * 4, DC), src, dst, zeros)
    h0, s1, s2 = _mlp_a4(h1, agg2, c2_W1, r(c2_b1))
    h2 = _mlp_b(h0, s1, s2, r(c2_g), r(c2_be), c2_W2, r(c2_b2))

    agg3 = _seg4(h2.reshape(N * 4, DC), src, dst, zeros)
    h0, s1, s2 = _mlp_a4(h2, agg3, c3_W1, r(c3_b1))
    h3 = _mlp_b(h0, s1, s2, r(c3_g), r(c3_be), c3_W2, r(c3_b2))

    return _final(h1, h2, h3, lin_W, r(lin_b))


# speedup vs baseline: 1.2225x; 1.2225x over previous
import functools

import jax
import jax.numpy as jnp
from jax import lax
from jax.experimental import pallas as pl
from jax.experimental.pallas import tpu as pltpu
from jax.experimental.pallas import tpu_sc as plsc

N = 10000
E = 160000
DIN = 256
DH = 512
DOUT = 128

DC = 128
NC = 2
NS = 16
EB = 80
NB = E // NS // EB
RPT = 640
NPAD = RPT * NS
TAIL = N - RPT * (NS - 1)

BN = 1000
GN = N // BN


def _make_seg_pair(C, p):
    mesh = plsc.VectorSubcoreMesh(core_axis_name="c", subcore_axis_name="s",
                                  num_cores=NC, num_subcores=NS)

    @functools.partial(
        pl.kernel,
        out_type=jax.ShapeDtypeStruct((NC, N, DC), jnp.float32),
        mesh=mesh,
        scratch_types=[
            pltpu.VMEM((NB, EB), jnp.int32),
            pltpu.VMEM((NB, EB), jnp.int32),
            pltpu.VMEM((EB,), jnp.int32),
            pltpu.VMEM((EB, DC), jnp.float32),
            pltpu.VMEM_SHARED((NPAD, DC), jnp.float32),
            pltpu.SemaphoreType.DMA,
        ],
    )
    def seg(x_hbm, src_hbm, dst_hbm, zeros_hbm, out_hbm,
            src_v, dst_v, sidx_v, rows_v, agg_sh, sem):
        cid = lax.axis_index("c")
        sid = lax.axis_index("s")
        c = 2 * p + cid
        full = pl.ds(sid * RPT, RPT)
        tail = pl.ds((NS - 1) * RPT, TAIL)
        pltpu.sync_copy(src_hbm.at[sid], src_v)
        pltpu.sync_copy(dst_hbm.at[sid], dst_v)

        @pl.when(sid < NS - 1)
        def _():
            pltpu.sync_copy(zeros_hbm, agg_sh.at[full])

        @pl.when(sid == NS - 1)
        def _():
            pltpu.sync_copy(zeros_hbm.at[pl.ds(0, TAIL)], agg_sh.at[tail])

        plsc.subcore_barrier()

        def body(j, carry):
            for k in range(EB // 16):
                sl = pl.ds(k * 16, 16)
                sidx_v[sl] = src_v[j, sl] * C + c
            pltpu.async_copy(x_hbm.at[sidx_v], rows_v, sem).wait()
            pltpu.sync_copy(rows_v, agg_sh.at[dst_v.at[j]], add=True)
            return carry

        lax.fori_loop(0, NB, body, 0)
        plsc.subcore_barrier()

        @pl.when(sid < NS - 1)
        def _():
            pltpu.sync_copy(agg_sh.at[full], out_hbm.at[cid].at[full])

        @pl.when(sid == NS - 1)
        def _():
            pltpu.sync_copy(agg_sh.at[tail], out_hbm.at[cid].at[tail])

        plsc.subcore_barrier()

    return seg


_seg2 = _make_seg_pair(2, 0)
_seg4a = _make_seg_pair(4, 0)
_seg4b = _make_seg_pair(4, 1)


def _mlp_a2_body(x_ref, agg_ref, w_ref, b_ref, h_ref, s1_ref, s2_ref):
    i = pl.program_id(0)
    agg = jnp.concatenate([agg_ref[0], agg_ref[1]], axis=-1)
    xa = x_ref[...] + agg
    h = jnp.dot(xa, w_ref[...], preferred_element_type=jnp.float32)
    h = h + b_ref[...]
    h_ref[...] = h

    @pl.when(i == 0)
    def _():
        s1_ref[...] = jnp.zeros_like(s1_ref)
        s2_ref[...] = jnp.zeros_like(s2_ref)

    s1_ref[...] += jnp.sum(h, axis=0, keepdims=True)
    s2_ref[...] += jnp.sum(h * h, axis=0, keepdims=True)


_mlp_a2 = pl.pallas_call(
    _mlp_a2_body,
    grid=(GN,),
    in_specs=[
        pl.BlockSpec((BN, DIN), lambda i: (i, 0)),
        pl.BlockSpec((NC, BN, DC), lambda i: (0, i, 0)),
        pl.BlockSpec((DIN, DH), lambda i: (0, 0)),
        pl.BlockSpec((1, DH), lambda i: (0, 0)),
    ],
    out_specs=[
        pl.BlockSpec((BN, DH), lambda i: (i, 0)),
        pl.BlockSpec((1, DH), lambda i: (0, 0)),
        pl.BlockSpec((1, DH), lambda i: (0, 0)),
    ],
    out_shape=[
        jax.ShapeDtypeStruct((N, DH), jnp.float32),
        jax.ShapeDtypeStruct((1, DH), jnp.float32),
        jax.ShapeDtypeStruct((1, DH), jnp.float32),
    ],
)


def _mlp_a4a_body(x_ref, agg_ref, w_ref, b_ref, h_ref):
    w = w_ref[...]
    h = jnp.dot(x_ref[...], w, preferred_element_type=jnp.float32)
    h = h + jnp.dot(agg_ref[0], w[0 * DC:1 * DC], preferred_element_type=jnp.float32)
    h = h + jnp.dot(agg_ref[1], w[1 * DC:2 * DC], preferred_element_type=jnp.float32)
    h_ref[...] = h + b_ref[...]


_mlp_a4a = pl.pallas_call(
    _mlp_a4a_body,
    grid=(GN,),
    in_specs=[
        pl.BlockSpec((BN, DH), lambda i: (i, 0)),
        pl.BlockSpec((NC, BN, DC), lambda i: (0, i, 0)),
        pl.BlockSpec((DH, DH), lambda i: (0, 0)),
        pl.BlockSpec((1, DH), lambda i: (0, 0)),
    ],
    out_specs=pl.BlockSpec((BN, DH), lambda i: (i, 0)),
    out_shape=jax.ShapeDtypeStruct((N, DH), jnp.float32),
)


def _mlp_a4b_body(hp_ref, agg_ref, w_ref, h_ref, s1_ref, s2_ref):
    i = pl.program_id(0)
    w = w_ref[...]
    h = hp_ref[...]
    h = h + jnp.dot(agg_ref[0], w[2 * DC:3 * DC], preferred_element_type=jnp.float32)
    h = h + jnp.dot(agg_ref[1], w[3 * DC:4 * DC], preferred_element_type=jnp.float32)
    h_ref[...] = h

    @pl.when(i == 0)
    def _():
        s1_ref[...] = jnp.zeros_like(s1_ref)
        s2_ref[...] = jnp.zeros_like(s2_ref)

    s1_ref[...] += jnp.sum(h, axis=0, keepdims=True)
    s2_ref[...] += jnp.sum(h * h, axis=0, keepdims=True)


_mlp_a4b = pl.pallas_call(
    _mlp_a4b_body,
    grid=(GN,),
    in_specs=[
        pl.BlockSpec((BN, DH), lambda i: (i, 0)),
        pl.BlockSpec((NC, BN, DC), lambda i: (0, i, 0)),
        pl.BlockSpec((DH, DH), lambda i: (0, 0)),
    ],
    out_specs=[
        pl.BlockSpec((BN, DH), lambda i: (i, 0)),
        pl.BlockSpec((1, DH), lambda i: (0, 0)),
        pl.BlockSpec((1, DH), lambda i: (0, 0)),
    ],
    out_shape=[
        jax.ShapeDtypeStruct((N, DH), jnp.float32),
        jax.ShapeDtypeStruct((1, DH), jnp.float32),
        jax.ShapeDtypeStruct((1, DH), jnp.float32),
    ],
)


def _mlp_b_body(h_ref, s1_ref, s2_ref, g_ref, be_ref, w_ref, b_ref, o_ref):
    mu = s1_ref[...] / N
    var = s2_ref[...] / N - mu * mu
    hn = (h_ref[...] - mu) * lax.rsqrt(var + 1e-5) * g_ref[...] + be_ref[...]
    hn = jnp.maximum(hn, 0.0)
    o = jnp.dot(hn, w_ref[...], preferred_element_type=jnp.float32)
    o_ref[...] = jnp.maximum(o + b_ref[...], 0.0)


_mlp_b = pl.pallas_call(
    _mlp_b_body,
    grid=(GN,),
    in_specs=[
        pl.BlockSpec((BN, DH), lambda i: (i, 0)),
        pl.BlockSpec((1, DH), lambda i: (0, 0)),
        pl.BlockSpec((1, DH), lambda i: (0, 0)),
        pl.BlockSpec((1, DH), lambda i: (0, 0)),
        pl.BlockSpec((1, DH), lambda i: (0, 0)),
        pl.BlockSpec((DH, DH), lambda i: (0, 0)),
        pl.BlockSpec((1, DH), lambda i: (0, 0)),
    ],
    out_specs=pl.BlockSpec((BN, DH), lambda i: (i, 0)),
    out_shape=jax.ShapeDtypeStruct((N, DH), jnp.float32),
)


def _final_body(h1_ref, h2_ref, h3_ref, w_ref, b_ref, o_ref):
    hcat = jnp.concatenate([h1_ref[...], h2_ref[...], h3_ref[...]], axis=-1)
    acc = jnp.dot(hcat, w_ref[...], preferred_element_type=jnp.float32)
    acc = acc + b_ref[...]
    m = jnp.max(acc, axis=1, keepdims=True)
    s = jnp.sum(jnp.exp(acc - m), axis=1, keepdims=True)
    o_ref[...] = acc - m - jnp.log(s)


_final = pl.pallas_call(
    _final_body,
    grid=(GN,),
    in_specs=[
        pl.BlockSpec((BN, DH), lambda i: (i, 0)),
        pl.BlockSpec((BN, DH), lambda i: (i, 0)),
        pl.BlockSpec((BN, DH), lambda i: (i, 0)),
        pl.BlockSpec((3 * DH, DOUT), lambda i: (0, 0)),
        pl.BlockSpec((1, DOUT), lambda i: (0, 0)),
    ],
    out_specs=pl.BlockSpec((BN, DOUT), lambda i: (i, 0)),
    out_shape=jax.ShapeDtypeStruct((N, DOUT), jnp.float32),
)


def kernel(x, edge_index, c1_W1, c1_b1, c1_g, c1_be, c1_W2, c1_b2,
           c2_W1, c2_b1, c2_g, c2_be, c2_W2, c2_b2,
           c3_W1, c3_b1, c3_g, c3_be, c3_W2, c3_b2, lin_W, lin_b):
    src = edge_index[0].reshape(NS, NB, EB)
    dst = edge_index[1].reshape(NS, NB, EB)
    zeros = jnp.zeros((RPT, DC), jnp.float32)
    r = lambda v: v.reshape(1, -1)

    agg1 = _seg2(x.reshape(N * 2, DC), src, dst, zeros)
    h0, s1, s2 = _mlp_a2(x, agg1, c1_W1, r(c1_b1))
    h1 = _mlp_b(h0, s1, s2, r(c1_g), r(c1_be), c1_W2, r(c1_b2))

    h1f = h1.reshape(N * 4, DC)
    a0 = _seg4a(h1f, src, dst, zeros)
    a1 = _seg4b(h1f, src, dst, zeros)
    hp = _mlp_a4a(h1, a0, c2_W1, r(c2_b1))
    h0, s1, s2 = _mlp_a4b(hp, a1, c2_W1)
    h2 = _mlp_b(h0, s1, s2, r(c2_g), r(c2_be), c2_W2, r(c2_b2))

    h2f = h2.reshape(N * 4, DC)
    a0 = _seg4a(h2f, src, dst, zeros)
    a1 = _seg4b(h2f, src, dst, zeros)
    hp = _mlp_a4a(h2, a0, c3_W1, r(c3_b1))
    h0, s1, s2 = _mlp_a4b(hp, a1, c3_W1)
    h3 = _mlp_b(h0, s1, s2, r(c3_g), r(c3_be), c3_W2, r(c3_b2))

    return _final(h1, h2, h3, lin_W, r(lin_b))


# speedup vs baseline: 1.2479x; 1.0207x over previous
import functools

import jax
import jax.numpy as jnp
from jax import lax
from jax.experimental import pallas as pl
from jax.experimental.pallas import tpu as pltpu
from jax.experimental.pallas import tpu_sc as plsc

N = 10000
E = 160000
DIN = 256
DH = 512
DOUT = 128

DC = 128
NC = 2
NS = 16
EB = 80
NB = E // NS // EB
RPT = 640
NPAD = RPT * NS
TAIL = N - RPT * (NS - 1)

BN = 1000
GN = N // BN


def _make_seg_pair(C, p):
    mesh = plsc.VectorSubcoreMesh(core_axis_name="c", subcore_axis_name="s",
                                  num_cores=NC, num_subcores=NS)

    @functools.partial(
        pl.kernel,
        out_type=jax.ShapeDtypeStruct((NC, N, DC), jnp.float32),
        mesh=mesh,
        scratch_types=[
            pltpu.VMEM((NB, EB), jnp.int32),
            pltpu.VMEM((NB, EB), jnp.int32),
            pltpu.VMEM((EB,), jnp.int32),
            pltpu.VMEM((EB, DC), jnp.float32),
            pltpu.VMEM_SHARED((NPAD, DC), jnp.float32),
            pltpu.SemaphoreType.DMA,
        ],
    )
    def seg(x_hbm, src_hbm, dst_hbm, zeros_hbm, out_hbm,
            src_v, dst_v, sidx_v, rows_v, agg_sh, sem):
        cid = lax.axis_index("c")
        sid = lax.axis_index("s")
        c = 2 * p + cid
        full = pl.ds(sid * RPT, RPT)
        tail = pl.ds((NS - 1) * RPT, TAIL)
        pltpu.sync_copy(src_hbm.at[sid], src_v)
        pltpu.sync_copy(dst_hbm.at[sid], dst_v)

        @pl.when(sid < NS - 1)
        def _():
            pltpu.sync_copy(zeros_hbm, agg_sh.at[full])

        @pl.when(sid == NS - 1)
        def _():
            pltpu.sync_copy(zeros_hbm.at[pl.ds(0, TAIL)], agg_sh.at[tail])

        plsc.subcore_barrier()

        def body(j, carry):
            for k in range(EB // 16):
                sl = pl.ds(k * 16, 16)
                sidx_v[sl] = src_v[j, sl] * C + c
            pltpu.async_copy(x_hbm.at[sidx_v], rows_v, sem).wait()
            pltpu.sync_copy(rows_v, agg_sh.at[dst_v.at[j]], add=True)
            return carry

        lax.fori_loop(0, NB, body, 0)
        plsc.subcore_barrier()

        @pl.when(sid < NS - 1)
        def _():
            pltpu.sync_copy(agg_sh.at[full], out_hbm.at[cid].at[full])

        @pl.when(sid == NS - 1)
        def _():
            pltpu.sync_copy(agg_sh.at[tail], out_hbm.at[cid].at[tail])

        plsc.subcore_barrier()

    return seg


_seg2 = _make_seg_pair(2, 0)
_seg4a = _make_seg_pair(4, 0)
_seg4b = _make_seg_pair(4, 1)


def _mlp_a2_body(x_ref, agg_ref, w_ref, b_ref, h_ref, s1_ref, s2_ref):
    i = pl.program_id(0)
    agg = jnp.concatenate([agg_ref[0], agg_ref[1]], axis=-1)
    xa = x_ref[...] + agg
    h = jnp.dot(xa, w_ref[...], preferred_element_type=jnp.float32)
    h = h + b_ref[...]
    h_ref[...] = h

    @pl.when(i == 0)
    def _():
        s1_ref[...] = jnp.zeros_like(s1_ref)
        s2_ref[...] = jnp.zeros_like(s2_ref)

    s1_ref[...] += jnp.sum(h, axis=0, keepdims=True)
    s2_ref[...] += jnp.sum(h * h, axis=0, keepdims=True)


_mlp_a2 = pl.pallas_call(
    _mlp_a2_body,
    grid=(GN,),
    in_specs=[
        pl.BlockSpec((BN, DIN), lambda i: (i, 0)),
        pl.BlockSpec((NC, BN, DC), lambda i: (0, i, 0)),
        pl.BlockSpec((DIN, DH), lambda i: (0, 0)),
        pl.BlockSpec((1, DH), lambda i: (0, 0)),
    ],
    out_specs=[
        pl.BlockSpec((BN, DH), lambda i: (i, 0)),
        pl.BlockSpec((1, DH), lambda i: (0, 0)),
        pl.BlockSpec((1, DH), lambda i: (0, 0)),
    ],
    out_shape=[
        jax.ShapeDtypeStruct((N, DH), jnp.float32),
        jax.ShapeDtypeStruct((1, DH), jnp.float32),
        jax.ShapeDtypeStruct((1, DH), jnp.float32),
    ],
)


def _mlp_a4_body(x_ref, agg0_ref, agg1_ref, w_ref, b_ref,
                 h_ref, s1_ref, s2_ref):
    i = pl.program_id(0)
    agg = jnp.concatenate(
        [agg0_ref[0], agg0_ref[1], agg1_ref[0], agg1_ref[1]], axis=-1)
    xa = x_ref[...] + agg
    h = jnp.dot(xa, w_ref[...], preferred_element_type=jnp.float32)
    h = h + b_ref[...]
    h_ref[...] = h

    @pl.when(i == 0)
    def _():
        s1_ref[...] = jnp.zeros_like(s1_ref)
        s2_ref[...] = jnp.zeros_like(s2_ref)

    s1_ref[...] += jnp.sum(h, axis=0, keepdims=True)
    s2_ref[...] += jnp.sum(h * h, axis=0, keepdims=True)


_mlp_a4 = pl.pallas_call(
    _mlp_a4_body,
    grid=(GN,),
    in_specs=[
        pl.BlockSpec((BN, DH), lambda i: (i, 0)),
        pl.BlockSpec((NC, BN, DC), lambda i: (0, i, 0)),
        pl.BlockSpec((NC, BN, DC), lambda i: (0, i, 0)),
        pl.BlockSpec((DH, DH), lambda i: (0, 0)),
        pl.BlockSpec((1, DH), lambda i: (0, 0)),
    ],
    out_specs=[
        pl.BlockSpec((BN, DH), lambda i: (i, 0)),
        pl.BlockSpec((1, DH), lambda i: (0, 0)),
        pl.BlockSpec((1, DH), lambda i: (0, 0)),
    ],
    out_shape=[
        jax.ShapeDtypeStruct((N, DH), jnp.float32),
        jax.ShapeDtypeStruct((1, DH), jnp.float32),
        jax.ShapeDtypeStruct((1, DH), jnp.float32),
    ],
)


def _mlp_b_body(h_ref, s1_ref, s2_ref, g_ref, be_ref, w_ref, b_ref, o_ref):
    mu = s1_ref[...] / N
    var = s2_ref[...] / N - mu * mu
    hn = (h_ref[...] - mu) * lax.rsqrt(var + 1e-5) * g_ref[...] + be_ref[...]
    hn = jnp.maximum(hn, 0.0)
    o = jnp.dot(hn, w_ref[...], preferred_element_type=jnp.float32)
    o_ref[...] = jnp.maximum(o + b_ref[...], 0.0)


_mlp_b = pl.pallas_call(
    _mlp_b_body,
    grid=(GN,),
    in_specs=[
        pl.BlockSpec((BN, DH), lambda i: (i, 0)),
        pl.BlockSpec((1, DH), lambda i: (0, 0)),
        pl.BlockSpec((1, DH), lambda i: (0, 0)),
        pl.BlockSpec((1, DH), lambda i: (0, 0)),
        pl.BlockSpec((1, DH), lambda i: (0, 0)),
        pl.BlockSpec((DH, DH), lambda i: (0, 0)),
        pl.BlockSpec((1, DH), lambda i: (0, 0)),
    ],
    out_specs=pl.BlockSpec((BN, DH), lambda i: (i, 0)),
    out_shape=jax.ShapeDtypeStruct((N, DH), jnp.float32),
)


def _final_body(h_ref, s1_ref, s2_ref, g_ref, be_ref, w2_ref, b2_ref,
                h1_ref, h2_ref, w_ref, b_ref, o_ref):
    mu = s1_ref[...] / N
    var = s2_ref[...] / N - mu * mu
    hn = (h_ref[...] - mu) * lax.rsqrt(var + 1e-5) * g_ref[...] + be_ref[...]
    hn = jnp.maximum(hn, 0.0)
    h3 = jnp.dot(hn, w2_ref[...], preferred_element_type=jnp.float32)
    h3 = jnp.maximum(h3 + b2_ref[...], 0.0)
    hcat = jnp.concatenate([h1_ref[...], h2_ref[...], h3], axis=-1)
    acc = jnp.dot(hcat, w_ref[...], preferred_element_type=jnp.float32)
    acc = acc + b_ref[...]
    m = jnp.max(acc, axis=1, keepdims=True)
    s = jnp.sum(jnp.exp(acc - m), axis=1, keepdims=True)
    o_ref[...] = acc - m - jnp.log(s)


_final = pl.pallas_call(
    _final_body,
    grid=(GN,),
    in_specs=[
        pl.BlockSpec((BN, DH), lambda i: (i, 0)),
        pl.BlockSpec((1, DH), lambda i: (0, 0)),
        pl.BlockSpec((1, DH), lambda i: (0, 0)),
        pl.BlockSpec((1, DH), lambda i: (0, 0)),
        pl.BlockSpec((1, DH), lambda i: (0, 0)),
        pl.BlockSpec((DH, DH), lambda i: (0, 0)),
        pl.BlockSpec((1, DH), lambda i: (0, 0)),
        pl.BlockSpec((BN, DH), lambda i: (i, 0)),
        pl.BlockSpec((BN, DH), lambda i: (i, 0)),
        pl.BlockSpec((3 * DH, DOUT), lambda i: (0, 0)),
        pl.BlockSpec((1, DOUT), lambda i: (0, 0)),
    ],
    out_specs=pl.BlockSpec((BN, DOUT), lambda i: (i, 0)),
    out_shape=jax.ShapeDtypeStruct((N, DOUT), jnp.float32),
)


def kernel(x, edge_index, c1_W1, c1_b1, c1_g, c1_be, c1_W2, c1_b2,
           c2_W1, c2_b1, c2_g, c2_be, c2_W2, c2_b2,
           c3_W1, c3_b1, c3_g, c3_be, c3_W2, c3_b2, lin_W, lin_b):
    src = edge_index[0].reshape(NS, NB, EB)
    dst = edge_index[1].reshape(NS, NB, EB)
    zeros = jnp.zeros((RPT, DC), jnp.float32)
    r = lambda v: v.reshape(1, -1)

    agg1 = _seg2(x.reshape(N * 2, DC), src, dst, zeros)
    h0, s1, s2 = _mlp_a2(x, agg1, c1_W1, r(c1_b1))
    h1 = _mlp_b(h0, s1, s2, r(c1_g), r(c1_be), c1_W2, r(c1_b2))

    h1f = h1.reshape(N * 4, DC)
    a0 = _seg4a(h1f, src, dst, zeros)
    a1 = _seg4b(h1f, src, dst, zeros)
    h0, s1, s2 = _mlp_a4(h1, a0, a1, c2_W1, r(c2_b1))
    h2 = _mlp_b(h0, s1, s2, r(c2_g), r(c2_be), c2_W2, r(c2_b2))

    h2f = h2.reshape(N * 4, DC)
    a0 = _seg4a(h2f, src, dst, zeros)
    a1 = _seg4b(h2f, src, dst, zeros)
    h0, s1, s2 = _mlp_a4(h2, a0, a1, c3_W1, r(c3_b1))
    return _final(h0, s1, s2, r(c3_g), r(c3_be), c3_W2, r(c3_b2),
                  h1, h2, lin_W, r(lin_b))


# speedup vs baseline: 1.2603x; 1.0099x over previous
import functools

import jax
import jax.numpy as jnp
from jax import lax
from jax.experimental import pallas as pl
from jax.experimental.pallas import tpu as pltpu
from jax.experimental.pallas import tpu_sc as plsc

N = 10000
E = 160000
DIN = 256
DH = 512
DOUT = 128

DC = 128
NC = 2
NS = 16
EB = 80
NB = E // NS // EB
RPT = 640
NPAD = RPT * NS
TAIL = N - RPT * (NS - 1)

BN = 1000
GN = N // BN


def _make_seg_sum(C):
    cpc = C // NC
    mesh = plsc.VectorSubcoreMesh(core_axis_name="c", subcore_axis_name="s",
                                  num_cores=NC, num_subcores=NS)

    @functools.partial(
        pl.kernel,
        out_type=jax.ShapeDtypeStruct((C, N, DC), jnp.float32),
        mesh=mesh,
        scratch_types=[
            pltpu.VMEM((NB, EB), jnp.int32),
            pltpu.VMEM((NB, EB), jnp.int32),
            pltpu.VMEM((EB,), jnp.int32),
            pltpu.VMEM((EB, DC), jnp.float32),
            pltpu.VMEM_SHARED((NPAD, DC), jnp.float32),
            pltpu.SemaphoreType.DMA,
        ],
    )
    def seg(x_hbm, src_hbm, dst_hbm, zeros_hbm, out_hbm,
            src_v, dst_v, sidx_v, rows_v, agg_sh, sem):
        cid = lax.axis_index("c")
        sid = lax.axis_index("s")
        full = pl.ds(sid * RPT, RPT)
        tail = pl.ds((NS - 1) * RPT, TAIL)
        pltpu.sync_copy(src_hbm.at[sid], src_v)
        pltpu.sync_copy(dst_hbm.at[sid], dst_v)

        for cc in range(cpc):
            c = cid * cpc + cc

            @pl.when(sid < NS - 1)
            def _():
                pltpu.sync_copy(zeros_hbm, agg_sh.at[full])

            @pl.when(sid == NS - 1)
            def _():
                pltpu.sync_copy(zeros_hbm.at[pl.ds(0, TAIL)], agg_sh.at[tail])

            plsc.subcore_barrier()

            def body(j, carry):
                for k in range(EB // 16):
                    sl = pl.ds(k * 16, 16)
                    sidx_v[sl] = src_v[j, sl] * C + c
                pltpu.async_copy(x_hbm.at[sidx_v], rows_v, sem).wait()
                pltpu.sync_copy(rows_v, agg_sh.at[dst_v.at[j]], add=True)
                return carry

            lax.fori_loop(0, NB, body, 0)
            plsc.subcore_barrier()

            @pl.when(sid < NS - 1)
            def _():
                pltpu.sync_copy(agg_sh.at[full], out_hbm.at[c].at[full])

            @pl.when(sid == NS - 1)
            def _():
                pltpu.sync_copy(agg_sh.at[tail], out_hbm.at[c].at[tail])

            plsc.subcore_barrier()

    return seg


_seg2 = _make_seg_sum(2)
_seg4 = _make_seg_sum(4)


def _mlp_a2_body(x_ref, agg_ref, w_ref, b_ref, h_ref, s1_ref, s2_ref):
    i = pl.program_id(0)
    agg = jnp.concatenate([agg_ref[0], agg_ref[1]], axis=-1)
    xa = x_ref[...] + agg
    h = jnp.dot(xa, w_ref[...], preferred_element_type=jnp.float32)
    h = h + b_ref[...]
    h_ref[...] = h

    @pl.when(i == 0)
    def _():
        s1_ref[...] = jnp.zeros_like(s1_ref)
        s2_ref[...] = jnp.zeros_like(s2_ref)

    s1_ref[...] += jnp.sum(h, axis=0, keepdims=True)
    s2_ref[...] += jnp.sum(h * h, axis=0, keepdims=True)


_mlp_a2 = pl.pallas_call(
    _mlp_a2_body,
    grid=(GN,),
    in_specs=[
        pl.BlockSpec((BN, DIN), lambda i: (i, 0)),
        pl.BlockSpec((NC, BN, DC), lambda i: (0, i, 0)),
        pl.BlockSpec((DIN, DH), lambda i: (0, 0)),
        pl.BlockSpec((1, DH), lambda i: (0, 0)),
    ],
    out_specs=[
        pl.BlockSpec((BN, DH), lambda i: (i, 0)),
        pl.BlockSpec((1, DH), lambda i: (0, 0)),
        pl.BlockSpec((1, DH), lambda i: (0, 0)),
    ],
    out_shape=[
        jax.ShapeDtypeStruct((N, DH), jnp.float32),
        jax.ShapeDtypeStruct((1, DH), jnp.float32),
        jax.ShapeDtypeStruct((1, DH), jnp.float32),
    ],
)


def _mlp_a4_body(x_ref, agg_ref, w_ref, b_ref, h_ref, s1_ref, s2_ref):
    i = pl.program_id(0)
    agg = jnp.concatenate(
        [agg_ref[0], agg_ref[1], agg_ref[2], agg_ref[3]], axis=-1)
    xa = x_ref[...] + agg
    h = jnp.dot(xa, w_ref[...], preferred_element_type=jnp.float32)
    h = h + b_ref[...]
    h_ref[...] = h

    @pl.when(i == 0)
    def _():
        s1_ref[...] = jnp.zeros_like(s1_ref)
        s2_ref[...] = jnp.zeros_like(s2_ref)

    s1_ref[...] += jnp.sum(h, axis=0, keepdims=True)
    s2_ref[...] += jnp.sum(h * h, axis=0, keepdims=True)


_mlp_a4 = pl.pallas_call(
    _mlp_a4_body,
    grid=(GN,),
    in_specs=[
        pl.BlockSpec((BN, DH), lambda i: (i, 0)),
        pl.BlockSpec((4, BN, DC), lambda i: (0, i, 0)),
        pl.BlockSpec((DH, DH), lambda i: (0, 0)),
        pl.BlockSpec((1, DH), lambda i: (0, 0)),
    ],
    out_specs=[
        pl.BlockSpec((BN, DH), lambda i: (i, 0)),
        pl.BlockSpec((1, DH), lambda i: (0, 0)),
        pl.BlockSpec((1, DH), lambda i: (0, 0)),
    ],
    out_shape=[
        jax.ShapeDtypeStruct((N, DH), jnp.float32),
        jax.ShapeDtypeStruct((1, DH), jnp.float32),
        jax.ShapeDtypeStruct((1, DH), jnp.float32),
    ],
)


def _mlp_b_body(h_ref, s1_ref, s2_ref, g_ref, be_ref, w_ref, b_ref, o_ref):
    mu = s1_ref[...] / N
    var = s2_ref[...] / N - mu * mu
    hn = (h_ref[...] - mu) * lax.rsqrt(var + 1e-5) * g_ref[...] + be_ref[...]
    hn = jnp.maximum(hn, 0.0)
    o = jnp.dot(hn, w_ref[...], preferred_element_type=jnp.float32)
    o_ref[...] = jnp.maximum(o + b_ref[...], 0.0)


_mlp_b = pl.pallas_call(
    _mlp_b_body,
    grid=(GN,),
    in_specs=[
        pl.BlockSpec((BN, DH), lambda i: (i, 0)),
        pl.BlockSpec((1, DH), lambda i: (0, 0)),
        pl.BlockSpec((1, DH), lambda i: (0, 0)),
        pl.BlockSpec((1, DH), lambda i: (0, 0)),
        pl.BlockSpec((1, DH), lambda i: (0, 0)),
        pl.BlockSpec((DH, DH), lambda i: (0, 0)),
        pl.BlockSpec((1, DH), lambda i: (0, 0)),
    ],
    out_specs=pl.BlockSpec((BN, DH), lambda i: (i, 0)),
    out_shape=jax.ShapeDtypeStruct((N, DH), jnp.float32),
)


def _final_body(h_ref, s1_ref, s2_ref, g_ref, be_ref, w2_ref, b2_ref,
                h1_ref, h2_ref, w_ref, b_ref, o_ref):
    mu = s1_ref[...] / N
    var = s2_ref[...] / N - mu * mu
    hn = (h_ref[...] - mu) * lax.rsqrt(var + 1e-5) * g_ref[...] + be_ref[...]
    hn = jnp.maximum(hn, 0.0)
    h3 = jnp.dot(hn, w2_ref[...], preferred_element_type=jnp.float32)
    h3 = jnp.maximum(h3 + b2_ref[...], 0.0)
    hcat = jnp.concatenate([h1_ref[...], h2_ref[...], h3], axis=-1)
    acc = jnp.dot(hcat, w_ref[...], preferred_element_type=jnp.float32)
    acc = acc + b_ref[...]
    m = jnp.max(acc, axis=1, keepdims=True)
    s = jnp.sum(jnp.exp(acc - m), axis=1, keepdims=True)
    o_ref[...] = acc - m - jnp.log(s)


_final = pl.pallas_call(
    _final_body,
    grid=(GN,),
    in_specs=[
        pl.BlockSpec((BN, DH), lambda i: (i, 0)),
        pl.BlockSpec((1, DH), lambda i: (0, 0)),
        pl.BlockSpec((1, DH), lambda i: (0, 0)),
        pl.BlockSpec((1, DH), lambda i: (0, 0)),
        pl.BlockSpec((1, DH), lambda i: (0, 0)),
        pl.BlockSpec((DH, DH), lambda i: (0, 0)),
        pl.BlockSpec((1, DH), lambda i: (0, 0)),
        pl.BlockSpec((BN, DH), lambda i: (i, 0)),
        pl.BlockSpec((BN, DH), lambda i: (i, 0)),
        pl.BlockSpec((3 * DH, DOUT), lambda i: (0, 0)),
        pl.BlockSpec((1, DOUT), lambda i: (0, 0)),
    ],
    out_specs=pl.BlockSpec((BN, DOUT), lambda i: (i, 0)),
    out_shape=jax.ShapeDtypeStruct((N, DOUT), jnp.float32),
)


def kernel(x, edge_index, c1_W1, c1_b1, c1_g, c1_be, c1_W2, c1_b2,
           c2_W1, c2_b1, c2_g, c2_be, c2_W2, c2_b2,
           c3_W1, c3_b1, c3_g, c3_be, c3_W2, c3_b2, lin_W, lin_b):
    src = edge_index[0].reshape(NS, NB, EB)
    dst = edge_index[1].reshape(NS, NB, EB)
    zeros = jnp.zeros((RPT, DC), jnp.float32)
    r = lambda v: v.reshape(1, -1)

    agg1 = _seg2(x.reshape(N * 2, DC), src, dst, zeros)
    h0, s1, s2 = _mlp_a2(x, agg1, c1_W1, r(c1_b1))
    h1 = _mlp_b(h0, s1, s2, r(c1_g), r(c1_be), c1_W2, r(c1_b2))

    agg2 = _seg4(h1.reshape(N * 4, DC), src, dst, zeros)
    h0, s1, s2 = _mlp_a4(h1, agg2, c2_W1, r(c2_b1))
    h2 = _mlp_b(h0, s1, s2, r(c2_g), r(c2_be), c2_W2, r(c2_b2))

    agg3 = _seg4(h2.reshape(N * 4, DC), src, dst, zeros)
    h0, s1, s2 = _mlp_a4(h2, agg3, c3_W1, r(c3_b1))
    return _final(h0, s1, s2, r(c3_g), r(c3_be), c3_W2, r(c3_b2),
                  h1, h2, lin_W, r(lin_b))


# speedup vs baseline: 1.2679x; 1.0061x over previous
import functools

import jax
import jax.numpy as jnp
from jax import lax
from jax.experimental import pallas as pl
from jax.experimental.pallas import tpu as pltpu
from jax.experimental.pallas import tpu_sc as plsc

N = 10000
E = 160000
DIN = 256
DH = 512
DOUT = 128

DC = 128
NC = 2
NS = 16
EB = 80
NB = E // NS // EB
RPT = 640
NPAD = RPT * NS
TAIL = N - RPT * (NS - 1)

BN = 2000
GN = N // BN


def _make_seg_sum(C):
    cpc = C // NC
    mesh = plsc.VectorSubcoreMesh(core_axis_name="c", subcore_axis_name="s",
                                  num_cores=NC, num_subcores=NS)

    @functools.partial(
        pl.kernel,
        out_type=jax.ShapeDtypeStruct((C, N, DC), jnp.float32),
        mesh=mesh,
        scratch_types=[
            pltpu.VMEM((NB, EB), jnp.int32),
            pltpu.VMEM((NB, EB), jnp.int32),
            pltpu.VMEM((EB,), jnp.int32),
            pltpu.VMEM((EB, DC), jnp.float32),
            pltpu.VMEM_SHARED((NPAD, DC), jnp.float32),
            pltpu.SemaphoreType.DMA,
        ],
    )
    def seg(x_hbm, src_hbm, dst_hbm, zeros_hbm, out_hbm,
            src_v, dst_v, sidx_v, rows_v, agg_sh, sem):
        cid = lax.axis_index("c")
        sid = lax.axis_index("s")
        full = pl.ds(sid * RPT, RPT)
        tail = pl.ds((NS - 1) * RPT, TAIL)
        pltpu.sync_copy(src_hbm.at[sid], src_v)
        pltpu.sync_copy(dst_hbm.at[sid], dst_v)

        for cc in range(cpc):
            c = cid * cpc + cc

            @pl.when(sid < NS - 1)
            def _():
                pltpu.sync_copy(zeros_hbm, agg_sh.at[full])

            @pl.when(sid == NS - 1)
            def _():
                pltpu.sync_copy(zeros_hbm.at[pl.ds(0, TAIL)], agg_sh.at[tail])

            plsc.subcore_barrier()

            def body(j, carry):
                for k in range(EB // 16):
                    sl = pl.ds(k * 16, 16)
                    sidx_v[sl] = src_v[j, sl] * C + c
                pltpu.async_copy(x_hbm.at[sidx_v], rows_v, sem).wait()
                pltpu.sync_copy(rows_v, agg_sh.at[dst_v.at[j]], add=True)
                return carry

            lax.fori_loop(0, NB, body, 0)
            plsc.subcore_barrier()

            @pl.when(sid < NS - 1)
            def _():
                pltpu.sync_copy(agg_sh.at[full], out_hbm.at[c].at[full])

            @pl.when(sid == NS - 1)
            def _():
                pltpu.sync_copy(agg_sh.at[tail], out_hbm.at[c].at[tail])

            plsc.subcore_barrier()

    return seg


_seg2 = _make_seg_sum(2)
_seg4 = _make_seg_sum(4)


def _mlp_a2_body(x_ref, agg_ref, w_ref, b_ref, h_ref, s1_ref, s2_ref):
    i = pl.program_id(0)
    agg = jnp.concatenate([agg_ref[0], agg_ref[1]], axis=-1)
    xa = x_ref[...] + agg
    h = jnp.dot(xa, w_ref[...], preferred_element_type=jnp.float32)
    h = h + b_ref[...]
    h_ref[...] = h

    @pl.when(i == 0)
    def _():
        s1_ref[...] = jnp.zeros_like(s1_ref)
        s2_ref[...] = jnp.zeros_like(s2_ref)

    s1_ref[...] += jnp.sum(h, axis=0, keepdims=True)
    s2_ref[...] += jnp.sum(h * h, axis=0, keepdims=True)


_mlp_a2 = pl.pallas_call(
    _mlp_a2_body,
    grid=(GN,),
    in_specs=[
        pl.BlockSpec((BN, DIN), lambda i: (i, 0)),
        pl.BlockSpec((NC, BN, DC), lambda i: (0, i, 0)),
        pl.BlockSpec((DIN, DH), lambda i: (0, 0)),
        pl.BlockSpec((1, DH), lambda i: (0, 0)),
    ],
    out_specs=[
        pl.BlockSpec((BN, DH), lambda i: (i, 0)),
        pl.BlockSpec((1, DH), lambda i: (0, 0)),
        pl.BlockSpec((1, DH), lambda i: (0, 0)),
    ],
    out_shape=[
        jax.ShapeDtypeStruct((N, DH), jnp.float32),
        jax.ShapeDtypeStruct((1, DH), jnp.float32),
        jax.ShapeDtypeStruct((1, DH), jnp.float32),
    ],
)


def _mlp_a4_body(x_ref, agg_ref, w_ref, b_ref, h_ref, s1_ref, s2_ref):
    i = pl.program_id(0)
    agg = jnp.concatenate(
        [agg_ref[0], agg_ref[1], agg_ref[2], agg_ref[3]], axis=-1)
    xa = x_ref[...] + agg
    h = jnp.dot(xa, w_ref[...], preferred_element_type=jnp.float32)
    h = h + b_ref[...]
    h_ref[...] = h

    @pl.when(i == 0)
    def _():
        s1_ref[...] = jnp.zeros_like(s1_ref)
        s2_ref[...] = jnp.zeros_like(s2_ref)

    s1_ref[...] += jnp.sum(h, axis=0, keepdims=True)
    s2_ref[...] += jnp.sum(h * h, axis=0, keepdims=True)


_mlp_a4 = pl.pallas_call(
    _mlp_a4_body,
    grid=(GN,),
    in_specs=[
        pl.BlockSpec((BN, DH), lambda i: (i, 0)),
        pl.BlockSpec((4, BN, DC), lambda i: (0, i, 0)),
        pl.BlockSpec((DH, DH), lambda i: (0, 0)),
        pl.BlockSpec((1, DH), lambda i: (0, 0)),
    ],
    out_specs=[
        pl.BlockSpec((BN, DH), lambda i: (i, 0)),
        pl.BlockSpec((1, DH), lambda i: (0, 0)),
        pl.BlockSpec((1, DH), lambda i: (0, 0)),
    ],
    out_shape=[
        jax.ShapeDtypeStruct((N, DH), jnp.float32),
        jax.ShapeDtypeStruct((1, DH), jnp.float32),
        jax.ShapeDtypeStruct((1, DH), jnp.float32),
    ],
)


def _mlp_b_body(h_ref, s1_ref, s2_ref, g_ref, be_ref, w_ref, b_ref, o_ref):
    mu = s1_ref[...] / N
    var = s2_ref[...] / N - mu * mu
    hn = (h_ref[...] - mu) * lax.rsqrt(var + 1e-5) * g_ref[...] + be_ref[...]
    hn = jnp.maximum(hn, 0.0)
    o = jnp.dot(hn, w_ref[...], preferred_element_type=jnp.float32)
    o_ref[...] = jnp.maximum(o + b_ref[...], 0.0)


_mlp_b = pl.pallas_call(
    _mlp_b_body,
    grid=(GN,),
    in_specs=[
        pl.BlockSpec((BN, DH), lambda i: (i, 0)),
        pl.BlockSpec((1, DH), lambda i: (0, 0)),
        pl.BlockSpec((1, DH), lambda i: (0, 0)),
        pl.BlockSpec((1, DH), lambda i: (0, 0)),
        pl.BlockSpec((1, DH), lambda i: (0, 0)),
        pl.BlockSpec((DH, DH), lambda i: (0, 0)),
        pl.BlockSpec((1, DH), lambda i: (0, 0)),
    ],
    out_specs=pl.BlockSpec((BN, DH), lambda i: (i, 0)),
    out_shape=jax.ShapeDtypeStruct((N, DH), jnp.float32),
)


def _final_body(h_ref, s1_ref, s2_ref, g_ref, be_ref, w2_ref, b2_ref,
                h1_ref, h2_ref, w_ref, b_ref, o_ref):
    mu = s1_ref[...] / N
    var = s2_ref[...] / N - mu * mu
    hn = (h_ref[...] - mu) * lax.rsqrt(var + 1e-5) * g_ref[...] + be_ref[...]
    hn = jnp.maximum(hn, 0.0)
    h3 = jnp.dot(hn, w2_ref[...], preferred_element_type=jnp.float32)
    h3 = jnp.maximum(h3 + b2_ref[...], 0.0)
    hcat = jnp.concatenate([h1_ref[...], h2_ref[...], h3], axis=-1)
    acc = jnp.dot(hcat, w_ref[...], preferred_element_type=jnp.float32)
    acc = acc + b_ref[...]
    m = jnp.max(acc, axis=1, keepdims=True)
    s = jnp.sum(jnp.exp(acc - m), axis=1, keepdims=True)
    o_ref[...] = acc - m - jnp.log(s)


_final = pl.pallas_call(
    _final_body,
    grid=(GN,),
    in_specs=[
        pl.BlockSpec((BN, DH), lambda i: (i, 0)),
        pl.BlockSpec((1, DH), lambda i: (0, 0)),
        pl.BlockSpec((1, DH), lambda i: (0, 0)),
        pl.BlockSpec((1, DH), lambda i: (0, 0)),
        pl.BlockSpec((1, DH), lambda i: (0, 0)),
        pl.BlockSpec((DH, DH), lambda i: (0, 0)),
        pl.BlockSpec((1, DH), lambda i: (0, 0)),
        pl.BlockSpec((BN, DH), lambda i: (i, 0)),
        pl.BlockSpec((BN, DH), lambda i: (i, 0)),
        pl.BlockSpec((3 * DH, DOUT), lambda i: (0, 0)),
        pl.BlockSpec((1, DOUT), lambda i: (0, 0)),
    ],
    out_specs=pl.BlockSpec((BN, DOUT), lambda i: (i, 0)),
    out_shape=jax.ShapeDtypeStruct((N, DOUT), jnp.float32),
)


def kernel(x, edge_index, c1_W1, c1_b1, c1_g, c1_be, c1_W2, c1_b2,
           c2_W1, c2_b1, c2_g, c2_be, c2_W2, c2_b2,
           c3_W1, c3_b1, c3_g, c3_be, c3_W2, c3_b2, lin_W, lin_b):
    src = edge_index[0].reshape(NS, NB, EB)
    dst = edge_index[1].reshape(NS, NB, EB)
    zeros = jnp.zeros((RPT, DC), jnp.float32)
    r = lambda v: v.reshape(1, -1)

    agg1 = _seg2(x.reshape(N * 2, DC), src, dst, zeros)
    h0, s1, s2 = _mlp_a2(x, agg1, c1_W1, r(c1_b1))
    h1 = _mlp_b(h0, s1, s2, r(c1_g), r(c1_be), c1_W2, r(c1_b2))

    agg2 = _seg4(h1.reshape(N * 4, DC), src, dst, zeros)
    h0, s1, s2 = _mlp_a4(h1, agg2, c2_W1, r(c2_b1))
    h2 = _mlp_b(h0, s1, s2, r(c2_g), r(c2_be), c2_W2, r(c2_b2))

    agg3 = _seg4(h2.reshape(N * 4, DC), src, dst, zeros)
    h0, s1, s2 = _mlp_a4(h2, agg3, c3_W1, r(c3_b1))
    return _final(h0, s1, s2, r(c3_g), r(c3_be), c3_W2, r(c3_b2),
                  h1, h2, lin_W, r(lin_b))
